# baseline (speedup 1.0000x reference)
import functools
import math

import jax
import jax.numpy as jnp
from jax import lax
from jax.experimental import pallas as pl
from jax.experimental.pallas import tpu as pltpu
from jax.experimental.pallas import tpu_sc as plsc

N = 10000
E = 320000
HID = 128
OUT = 64
H = 8
DH = 16
L = 2

NS = 16
EPW = E // NS
B = 32
NB = EPW // B
NZ = 10
RPS = N // NZ

_f32 = jnp.float32

_HEAD_AT_LANE = (0, 0, 4, 4, 2, 2, 6, 6, 1, 1, 5, 5, 3, 3, 7, 7)
_LANE_OF = (0, 8, 4, 12, 2, 10, 6, 14)


_RB = 1000
_G = (2 * N) // _RB


def _in_proj_body(x_ref, w_ref, b_ref, o_ref):
    y = jnp.dot(x_ref[...], w_ref[0], preferred_element_type=_f32) + b_ref[0]
    o_ref[...] = jnp.maximum(y, 0.0)


def _qkv_body(x_ref, wq_ref, bq_ref, wk_ref, bk_ref, wv_ref, bv_ref,
              q_ref, k_ref, v_ref):
    x = x_ref[...]
    q_ref[...] = jnp.dot(x, wq_ref[0], preferred_element_type=_f32) + bq_ref[0]
    k_ref[...] = jnp.dot(x, wk_ref[0], preferred_element_type=_f32) + bk_ref[0]
    v_ref[...] = jnp.dot(x, wv_ref[0], preferred_element_type=_f32) + bv_ref[0]


def _finish_body(msg_ref, den_ref, x_ref, aw_ref, ab_ref, beta_ref, rmat_ref,
                 o_ref):
    num = msg_ref[0]
    den16 = den_ref[0][:, :16]
    den = jnp.dot(den16, rmat_ref[...], preferred_element_type=_f32) + 1e-16
    z = jax.nn.gelu(num / den)
    o = jnp.dot(z, aw_ref[0], preferred_element_type=_f32) + ab_ref[0]
    beta = jnp.where(pl.program_id(0) >= 10, beta_ref[1], beta_ref[0])
    o_ref[...] = jnp.maximum(beta * o + (1.0 - beta) * x_ref[...], 0.0)


def _out_proj_body(x_ref, w_ref, b_ref, o_ref):
    o_ref[...] = (jnp.dot(x_ref[...], w_ref[...], preferred_element_type=_f32)
                  + b_ref[...][None, :])


def _typed_w(i):
    return (i // 10, 0, 0)


def _typed_b(i):
    return (i // 10, 0, 0)


def _in_proj(x, w, b):
    return pl.pallas_call(
        _in_proj_body,
        grid=(_G,),
        in_specs=[
            pl.BlockSpec((_RB, HID), lambda i: (i, 0)),
            pl.BlockSpec((1, HID, HID), _typed_w),
            pl.BlockSpec((1, 1, HID), _typed_b),
        ],
        out_specs=pl.BlockSpec((_RB, HID), lambda i: (i, 0)),
        out_shape=jax.ShapeDtypeStruct((2 * N, HID), _f32),
    )(x, w, b)


def _qkv(x, wq, bq, wk, bk, wv, bv):
    row = pl.BlockSpec((_RB, HID), lambda i: (i, 0))
    return pl.pallas_call(
        _qkv_body,
        grid=(_G,),
        in_specs=[row] + [pl.BlockSpec((1, HID, HID), _typed_w),
                          pl.BlockSpec((1, 1, HID), _typed_b)] * 3,
        out_specs=[row, row, row],
        out_shape=[jax.ShapeDtypeStruct((2 * N, HID), _f32)] * 3,
    )(x, wq, bq, wk, bk, wv, bv)


def _finish(msg, den, x, aw, ab, beta, rmat):
    def agg_map(i):
        return ((i // 10 + 1) % 2, i % 10, 0)

    return pl.pallas_call(
        _finish_body,
        grid=(_G,),
        in_specs=[
            pl.BlockSpec((1, _RB, HID), agg_map),
            pl.BlockSpec((1, _RB, HID), agg_map),
            pl.BlockSpec((_RB, HID), lambda i: (i, 0)),
            pl.BlockSpec((1, HID, HID), _typed_w),
            pl.BlockSpec((1, 1, HID), _typed_b),
            pl.BlockSpec((2,), lambda i: (0,), memory_space=pltpu.SMEM),
            pl.BlockSpec((16, HID), lambda i: (0, 0)),
        ],
        out_specs=pl.BlockSpec((_RB, HID), lambda i: (i, 0)),
        out_shape=jax.ShapeDtypeStruct((2 * N, HID), _f32),
    )(msg, den, x, aw, ab, beta, rmat)


def _out_proj(x, w, b):
    return pl.pallas_call(
        _out_proj_body,
        grid=(_G,),
        in_specs=[
            pl.BlockSpec((_RB, HID), lambda i: (i, 0)),
            pl.BlockSpec((HID, OUT), lambda i: (0, 0)),
            pl.BlockSpec((OUT,), lambda i: (0,)),
        ],
        out_specs=pl.BlockSpec((_RB, OUT), lambda i: (i, 0)),
        out_shape=jax.ShapeDtypeStruct((2 * N, OUT), _f32),
    )(x, w, b)


def _sc_body(krel_hbm, vrel_hbm, q_hbm, esd_hbm, zeros_hbm,
             msg_out, den_out,
             ibuf, dadj, sidx, q_rows, k_rows, v_rows, denb,
             acc_msg, acc_den,
             sem_i0, sem_i1, sem_i2, sem_g0, sem_g1, sem_g2,
             sem_s0, sem_s1, sem_s2):
    c = lax.axis_index("c")
    s = lax.axis_index("s")
    zero16 = jnp.zeros((16,), _f32)
    lanes = lax.iota(jnp.int32, 16)
    sem_i = (sem_i0, sem_i1, sem_i2)
    sem_g = (sem_g0, sem_g1, sem_g2)
    sem_s = (sem_s0, sem_s1, sem_s2)

    r0 = s * RPS

    @pl.when(s < NZ)
    def _zero_acc():
        pltpu.sync_copy(zeros_hbm, acc_msg.at[pl.ds(r0, RPS), :])
        pltpu.sync_copy(zeros_hbm.at[:, pl.ds(0, 16)],
                        acc_den.at[pl.ds(r0, RPS), :])

    plsc.subcore_barrier()

    ebase = c * E + s * EPW
    src_off = c * N
    q_off = (1 - c) * N

    def idx_desc(j, b):
        off = ebase + j * B
        return pltpu.make_async_copy(esd_hbm.at[:, pl.ds(off, B)],
                                     ibuf.at[b], sem_i[b])

    def gather_descs(b):
        return (pltpu.make_async_copy(krel_hbm.at[ibuf.at[b, 0]],
                                      k_rows.at[b], sem_g[b]),
                pltpu.make_async_copy(vrel_hbm.at[ibuf.at[b, 0]],
                                      v_rows.at[b], sem_g[b]),
                pltpu.make_async_copy(q_hbm.at[dadj.at[b]],
                                      q_rows.at[b], sem_g[b]))

    def scatter_drain_descs(b):
        return (pltpu.make_async_copy(v_rows.at[b], acc_msg.at[sidx.at[b]],
                                      sem_s[b]),
                pltpu.make_async_copy(denb.at[b], acc_den.at[sidx.at[b]],
                                      sem_s[b]))

    def adjust_and_gather(b):
        for kk in range(B // 16):
            sl = pl.ds(16 * kk, 16)
            ibuf[b, 0, sl] = ibuf[b, 0, sl] + src_off
            dadj[b, sl] = ibuf[b, 1, sl] + q_off
        for d in gather_descs(b):
            d.start()

    def _proc(j, b):
        bn = (b + 1) % 3
        bp = (b + 2) % 3
        jt = jnp.int32(j)
        for d in gather_descs(b):
            d.wait()
        for kk in range(B // 16):
            sl = pl.ds(16 * kk, 16)
            sidx[b, sl] = ibuf[b, 1, sl]

        @pl.when(jt + 1 < NB)
        def _issue_next():
            idx_desc(j + 1, bn).wait()

            @pl.when(jt >= 2)
            def _drain_prev_scatter():
                for d in scatter_drain_descs(bn):
                    d.wait()

            adjust_and_gather(bn)

        @pl.when(jt + 2 < NB)
        def _prefetch_idx():
            idx_desc(j + 2, bp).start()

        def _one_edge(e):
            def shuf(x, m):
                return x.at[lanes ^ m].get(mode="promise_in_bounds")

            u8 = []
            for h in range(H):
                sl = pl.ds(16 * h, 16)
                prod = q_rows[b, e, sl] * k_rows[b, e, sl]
                u8.append(prod + shuf(prod, 8))
            c4 = [jnp.where(lanes < 8, u8[2 * i], u8[2 * i + 1]) for i in range(4)]
            u4 = [x + shuf(x, 4) for x in c4]
            c2 = [jnp.where((lanes & 4) == 0, u4[2 * i], u4[2 * i + 1])
                  for i in range(2)]
            u2 = [x + shuf(x, 2) for x in c2]
            f1 = jnp.where((lanes & 2) == 0, u2[0], u2[1])
            g = f1 + shuf(f1, 1)
            ex = jnp.exp(g)
            denb[b, e, pl.ds(0, 16)] = ex
            for h in range(H):
                sl = pl.ds(16 * h, 16)
                v_rows[b, e, sl] = v_rows[b, e, sl] * ex[_LANE_OF[h]]

        def _edge(i, ecarry):
            _one_edge(2 * i)
            _one_edge(2 * i + 1)
            return ecarry

        lax.fori_loop(0, B // 2, _edge, 0)
        pltpu.async_copy(v_rows.at[b], acc_msg.at[sidx.at[b]], sem_s[b],
                         add=True)
        pltpu.async_copy(denb.at[b], acc_den.at[sidx.at[b]], sem_s[b],
                         add=True)

    idx_desc(0, 0).start()
    idx_desc(1, 1).start()
    idx_desc(0, 0).wait()
    adjust_and_gather(0)

    def _triple(jj, carry):
        for b in range(3):
            _proc(3 * jj + b, b)
        return carry

    lax.fori_loop(0, NB // 3, _triple, 0)
    _proc(NB - 1, (NB - 1) % 3)
    for t in (NB - 3, NB - 2, NB - 1):
        for d in scatter_drain_descs(t % 3):
            d.wait()
    plsc.subcore_barrier()

    @pl.when(s < NZ)
    def _copy_out():
        pltpu.sync_copy(acc_msg.at[pl.ds(r0, RPS), :],
                        msg_out.at[c, pl.ds(r0, RPS), :])
        pltpu.sync_copy(acc_den.at[pl.ds(r0, RPS), :],
                        den_out.at[c, pl.ds(r0, RPS), pl.ds(0, 16)])


def _sc_agg(krel_cat, vrel_cat, q_cat, esd, zeros_pad):
    fn = pl.kernel(
        _sc_body,
        out_type=(jax.ShapeDtypeStruct((2, N, HID), _f32),
                  jax.ShapeDtypeStruct((2, N, HID), _f32)),
        mesh=plsc.VectorSubcoreMesh(core_axis_name="c", subcore_axis_name="s"),
        compiler_params=pltpu.CompilerParams(use_tc_tiling_on_sc=False),
        scratch_types=[
            pltpu.VMEM((3, 2, B), jnp.int32),
            pltpu.VMEM((3, B), jnp.int32),
            pltpu.VMEM((3, B), jnp.int32),
            pltpu.VMEM((3, B, HID), _f32),
            pltpu.VMEM((3, B, HID), _f32),
            pltpu.VMEM((3, B, HID), _f32),
            pltpu.VMEM((3, B, 16), _f32),
            pltpu.VMEM_SHARED((N, HID), _f32),
            pltpu.VMEM_SHARED((N, 16), _f32),
        ] + [pltpu.SemaphoreType.DMA] * 9,
    )
    return fn(krel_cat, vrel_cat, q_cat, esd, zeros_pad)


def _merge_rel(w, b, rel):
    wm = jnp.einsum("ihd,hde->ihe", w.reshape(HID, H, DH), rel).reshape(HID, HID)
    bm = jnp.einsum("hd,hde->he", b.reshape(H, DH), rel).reshape(HID)
    return wm, bm


def kernel(x_user, x_job, edge_uj, edge_ju, params):
    p = params
    x_cat = jnp.concatenate([x_user, x_job], axis=0)
    esd = jnp.concatenate([edge_uj, edge_ju], axis=1).astype(jnp.int32)
    zeros_pad = jnp.zeros((RPS, HID), _f32)

    rmat = 0.5 * jnp.kron(
        jax.nn.one_hot(jnp.asarray(_HEAD_AT_LANE), H, dtype=_f32),
        jnp.ones((1, DH), _f32))

    w_in = jnp.stack([p["in_user_w"], p["in_job_w"]])
    b_in = jnp.stack([p["in_user_b"], p["in_job_b"]])[:, None, :]
    y = _in_proj(x_cat, w_in, b_in)

    for l in range(L):
        scale = 1.0 / math.sqrt(DH)
        arel_u = p[f"l{l}_arel_uj"] * (p[f"l{l}_prel_uj"] * scale)[:, None, None]
        arel_j = p[f"l{l}_arel_ju"] * (p[f"l{l}_prel_ju"] * scale)[:, None, None]
        wk_u, bk_u = _merge_rel(p[f"l{l}_K_user_w"], p[f"l{l}_K_user_b"], arel_u)
        wk_j, bk_j = _merge_rel(p[f"l{l}_K_job_w"], p[f"l{l}_K_job_b"], arel_j)
        wv_u, bv_u = _merge_rel(p[f"l{l}_V_user_w"], p[f"l{l}_V_user_b"],
                                p[f"l{l}_mrel_uj"])
        wv_j, bv_j = _merge_rel(p[f"l{l}_V_job_w"], p[f"l{l}_V_job_b"],
                                p[f"l{l}_mrel_ju"])
        wq = jnp.stack([p[f"l{l}_Q_user_w"], p[f"l{l}_Q_job_w"]])
        bq = jnp.stack([p[f"l{l}_Q_user_b"], p[f"l{l}_Q_job_b"]])[:, None, :]
        wk = jnp.stack([wk_u, wk_j])
        bk = jnp.stack([bk_u, bk_j])[:, None, :]
        wv = jnp.stack([wv_u, wv_j])
        bv = jnp.stack([bv_u, bv_j])[:, None, :]

        q_cat, krel_cat, vrel_cat = _qkv(y, wq, bq, wk, bk, wv, bv)
        msg, den = _sc_agg(krel_cat, vrel_cat, q_cat, esd, zeros_pad)

        aw = jnp.stack([p[f"l{l}_A_user_w"], p[f"l{l}_A_job_w"]])
        ab = jnp.stack([p[f"l{l}_A_user_b"], p[f"l{l}_A_job_b"]])[:, None, :]
        beta = jax.nn.sigmoid(jnp.stack([p[f"l{l}_skip_user"],
                                         p[f"l{l}_skip_job"]]))
        y = _finish(msg, den, y, aw, ab, beta, rmat)

    out = _out_proj(y, p["out_w"], p["out_b"])
    return out[:N], out[N:]
